# baseline (speedup 1.0000x reference)
import jax
import jax.numpy as jnp
from jax.experimental import pallas as pl
from jax.experimental.pallas import tpu as pltpu

_R, _C = 2048, 8192
_BR = 256


def _minmax_body(x_ref, mn_ref, mx_ref, acc_ref):
    i = pl.program_id(0)
    n = pl.num_programs(0)
    bmin = jnp.min(x_ref[...])
    bmax = jnp.max(x_ref[...])

    @pl.when(i == 0)
    def _():
        acc_ref[0] = bmin
        acc_ref[1] = bmax

    @pl.when(i > 0)
    def _():
        acc_ref[0] = jnp.minimum(acc_ref[0], bmin)
        acc_ref[1] = jnp.maximum(acc_ref[1], bmax)

    @pl.when(i == n - 1)
    def _():
        mn_ref[0] = acc_ref[0]
        mx_ref[0] = acc_ref[1]


def _quant_body(s_ref, x_ref, o_ref):
    mn = s_ref[0]
    sc = s_ref[1]
    o_ref[...] = jnp.round((x_ref[...] - mn) / sc) * sc + mn


def kernel(tensor, bits):
    x = tensor.reshape(_R, _C)

    mn, mx = pl.pallas_call(
        _minmax_body,
        grid=(_R // _BR,),
        in_specs=[pl.BlockSpec((_BR, _C), lambda i: (i, 0))],
        out_specs=[
            pl.BlockSpec(memory_space=pltpu.SMEM),
            pl.BlockSpec(memory_space=pltpu.SMEM),
        ],
        out_shape=[
            jax.ShapeDtypeStruct((1,), jnp.float32),
            jax.ShapeDtypeStruct((1,), jnp.float32),
        ],
        scratch_shapes=[pltpu.SMEM((2,), jnp.float32)],
    )(x)

    min_val = mn[0]
    scale = (mx[0] - min_val) / (2 ** bits - 1)
    s = jnp.stack([min_val, scale])

    y = pl.pallas_call(
        _quant_body,
        grid=(_R // _BR,),
        in_specs=[
            pl.BlockSpec(memory_space=pltpu.SMEM),
            pl.BlockSpec((_BR, _C), lambda i: (i, 0)),
        ],
        out_specs=pl.BlockSpec((_BR, _C), lambda i: (i, 0)),
        out_shape=jax.ShapeDtypeStruct((_R, _C), jnp.float32),
    )(s, x)

    return y.reshape(tensor.shape)
